# Initial kernel scaffold; baseline (speedup 1.0000x reference)
import jax
import jax.numpy as jnp
from jax.experimental import pallas as pl


def kernel(xyz, new_xyz, features):
    raise NotImplementedError("write your pallas kernel here")


# speedup vs baseline: 12.2688x; 12.2688x over previous
import functools

import jax
import jax.numpy as jnp
from jax import lax
from jax.experimental import pallas as pl
from jax.experimental.pallas import tpu as pltpu
from jax.experimental.pallas import tpu_sc as plsc

B = 4
N = 8192
NPOINT = 1024
NSAMPLE = 32
C = 128
R2 = 0.2 * 0.2

NC = 2
NS = 16
L = 16
NW = NC * NS
CPT = (B * NPOINT) // NW
TPB = NW // B
CHT = C // TPB
NCHUNK = N // L

_mesh = plsc.VectorSubcoreMesh(
    core_axis_name="c", subcore_axis_name="s", num_cores=NC, num_subcores=NS)


def _bf16_round(x):
    bits = plsc.bitcast(x, jnp.int32)
    lsb = (bits >> 16) & 1
    rb = (bits + 0x7FFF + lsb) & jnp.int32(-65536)
    return plsc.bitcast(rb, jnp.float32)


@functools.partial(
    pl.kernel,
    out_type=(
        jax.ShapeDtypeStruct((B, NPOINT, NSAMPLE), jnp.int32),
        jax.ShapeDtypeStruct((B, 3, NPOINT, NSAMPLE), jnp.float32),
    ),
    mesh=_mesh,
    compiler_params=pltpu.CompilerParams(use_tc_tiling_on_sc=False, needs_layout_passes=False),
    scratch_types=[
        pltpu.VMEM((N,), jnp.float32),
        pltpu.VMEM((N,), jnp.float32),
        pltpu.VMEM((N,), jnp.float32),
        pltpu.VMEM((CPT,), jnp.float32),
        pltpu.VMEM((CPT,), jnp.float32),
        pltpu.VMEM((CPT,), jnp.float32),
        pltpu.VMEM((CPT, NSAMPLE), jnp.int32),
        pltpu.VMEM((3, CPT, NSAMPLE), jnp.float32),
        pltpu.VMEM((N,), jnp.float32),
        pltpu.VMEM((N,), jnp.float32),
        pltpu.VMEM((N,), jnp.float32),
    ],
)
def _ball_query_kernel(xyzt, newt, idx_out, gxyz_out,
                       px, py, pz, cxs, cys, czs, selbuf, xyzbuf,
                       pxb, pyb, pzb):
    ci = lax.axis_index("c")
    si = lax.axis_index("s")
    wid = ci * NS + si
    b = wid // TPB
    m0 = (wid % TPB) * CPT

    pltpu.sync_copy(xyzt.at[b, 0, pl.ds(0, N)], px)
    pltpu.sync_copy(xyzt.at[b, 1, pl.ds(0, N)], py)
    pltpu.sync_copy(xyzt.at[b, 2, pl.ds(0, N)], pz)
    pltpu.sync_copy(newt.at[b, 0, pl.ds(m0, CPT)], cxs)
    pltpu.sync_copy(newt.at[b, 1, pl.ds(m0, CPT)], cys)
    pltpu.sync_copy(newt.at[b, 2, pl.ds(m0, CPT)], czs)

    iot = lax.iota(jnp.int32, L)
    zz = jnp.zeros((L,), jnp.int32)
    for w in range(CPT // L):
        plsc.store_scatter(selbuf, [w * L + iot, zz], zz)

    def pre_round(i, carry):
        base = i * L
        pxb[pl.ds(base, L)] = _bf16_round(px[pl.ds(base, L)])
        pyb[pl.ds(base, L)] = _bf16_round(py[pl.ds(base, L)])
        pzb[pl.ds(base, L)] = _bf16_round(pz[pl.ds(base, L)])
        return carry

    lax.fori_loop(0, NCHUNK, pre_round, 0)

    def per_center(mi, carry):
        msp = jnp.full((L,), mi, jnp.int32)
        cx = plsc.load_gather(cxs, [msp])
        cy = plsc.load_gather(cys, [msp])
        cz = plsc.load_gather(czs, [msp])
        sc = (cx * cx + cy * cy) + cz * cz
        cxb = _bf16_round(cx)
        cyb = _bf16_round(cy)
        czb = _bf16_round(cz)

        def cond(st):
            i, cnt = st
            return jnp.logical_and(i < NCHUNK, cnt < NSAMPLE)

        def body(st):
            i, cnt = st
            base = i * L
            xs = px[pl.ds(base, L)]
            ys = py[pl.ds(base, L)]
            zs = pz[pl.ds(base, L)]
            sp = (xs * xs + ys * ys) + zs * zs
            dot = (cxb * pxb[pl.ds(base, L)] + cyb * pyb[pl.ds(base, L)]) \
                + czb * pzb[pl.ds(base, L)]
            d2 = (sc + sp) - 2.0 * dot
            m = d2 < R2
            mi32 = m.astype(jnp.int32)
            inc = plsc.cumsum(mi32)
            cntv = jnp.full((L,), cnt, jnp.int32)
            slot = (cntv + inc) - 1
            wm = jnp.logical_and(m, slot < NSAMPLE)
            plsc.store_scatter(selbuf, [msp, slot], base + iot, mask=wm)
            return (i + 1, cnt + jnp.sum(mi32))

        _, cnt = lax.while_loop(cond, body, (jnp.int32(0), jnp.int32(0)))

        firstv = plsc.load_gather(selbuf, [msp, zz])
        cntv = jnp.full((L,), cnt, jnp.int32)
        for h in range(NSAMPLE // L):
            jv = h * L + iot
            cur = plsc.load_gather(selbuf, [msp, jv])
            selv = jnp.where(jv >= cntv, firstv, cur)
            plsc.store_scatter(selbuf, [msp, jv], selv)
            gx = plsc.load_gather(px, [selv]) - cx
            gy = plsc.load_gather(py, [selv]) - cy
            gz = plsc.load_gather(pz, [selv]) - cz
            plsc.store_scatter(xyzbuf, [zz, msp, jv], gx)
            plsc.store_scatter(xyzbuf, [zz + 1, msp, jv], gy)
            plsc.store_scatter(xyzbuf, [zz + 2, msp, jv], gz)
        return carry

    lax.fori_loop(0, CPT, per_center, 0)

    pltpu.sync_copy(selbuf, idx_out.at[b, pl.ds(m0, CPT), pl.ds(0, NSAMPLE)])
    pltpu.sync_copy(xyzbuf, gxyz_out.at[b, pl.ds(0, 3), pl.ds(m0, CPT), pl.ds(0, NSAMPLE)])


@functools.partial(
    pl.kernel,
    out_type=jax.ShapeDtypeStruct((B, 3 + C, NPOINT, NSAMPLE), jnp.float32),
    mesh=_mesh,
    compiler_params=pltpu.CompilerParams(use_tc_tiling_on_sc=False, needs_layout_passes=False),
    scratch_types=[
        pltpu.VMEM((NPOINT, NSAMPLE), jnp.int32),
        pltpu.VMEM((N,), jnp.float32),
        pltpu.VMEM((NPOINT, NSAMPLE), jnp.float32),
    ],
)
def _group_kernel(features, idxq, gxyz, out, idx_s, frow, orow):
    ci = lax.axis_index("c")
    si = lax.axis_index("s")
    wid = ci * NS + si
    b = wid // TPB
    c0 = (wid % TPB) * CHT

    pltpu.sync_copy(idxq.at[b, pl.ds(0, NPOINT), pl.ds(0, NSAMPLE)], idx_s)

    @pl.when(wid % TPB == 0)
    def _copy_xyz():
        pltpu.sync_copy(gxyz.at[b, pl.ds(0, 3), pl.ds(0, NPOINT), pl.ds(0, NSAMPLE)],
                        out.at[b, pl.ds(0, 3), pl.ds(0, NPOINT), pl.ds(0, NSAMPLE)])

    iot = lax.iota(jnp.int32, L)

    def per_channel(k, carry):
        ch = c0 + k
        pltpu.sync_copy(features.at[b, ch, pl.ds(0, N)], frow)

        def per_row(ri, carry2):
            rv = jnp.full((L,), ri, jnp.int32)
            for h in range(NSAMPLE // L):
                cv = h * L + iot
                iv = plsc.load_gather(idx_s, [rv, cv])
                vals = plsc.load_gather(frow, [iv])
                plsc.store_scatter(orow, [rv, cv], vals)
            return carry2

        lax.fori_loop(0, NPOINT, per_row, 0)
        pltpu.sync_copy(orow, out.at[b, 3 + ch, pl.ds(0, NPOINT), pl.ds(0, NSAMPLE)])
        return carry

    lax.fori_loop(0, CHT, per_channel, 0)


def kernel(xyz, new_xyz, features):
    xyzt = jnp.transpose(xyz, (0, 2, 1))
    newt = jnp.transpose(new_xyz, (0, 2, 1))
    idxq, gxyz = _ball_query_kernel(xyzt, newt)
    return _group_kernel(features, idxq, gxyz)


# speedup vs baseline: 16.4078x; 1.3374x over previous
import functools

import jax
import jax.numpy as jnp
from jax import lax
from jax.experimental import pallas as pl
from jax.experimental.pallas import tpu as pltpu
from jax.experimental.pallas import tpu_sc as plsc

B = 4
N = 8192
NPOINT = 1024
NSAMPLE = 32
C = 128
R2 = 0.2 * 0.2

NC = 2
NS = 16
L = 16
NW = NC * NS
CPT = (B * NPOINT) // NW
TPB = NW // B
CHT = C // TPB
NCHUNK = N // L
CB = 4
RB = 256

_mesh = plsc.VectorSubcoreMesh(
    core_axis_name="c", subcore_axis_name="s", num_cores=NC, num_subcores=NS)
_params = pltpu.CompilerParams(use_tc_tiling_on_sc=False,
                               needs_layout_passes=False)


def _bf16_round(x):
    bits = plsc.bitcast(x, jnp.int32)
    lsb = (bits >> 16) & 1
    rb = (bits + 0x7FFF + lsb) & jnp.int32(-65536)
    return plsc.bitcast(rb, jnp.float32)


@functools.partial(
    pl.kernel,
    out_type=(
        jax.ShapeDtypeStruct((B, NPOINT, NSAMPLE), jnp.int32),
        jax.ShapeDtypeStruct((B, 3, NPOINT, NSAMPLE), jnp.float32),
    ),
    mesh=_mesh,
    compiler_params=_params,
    scratch_types=[
        pltpu.VMEM((N, 3), jnp.float32),
        pltpu.VMEM((N,), jnp.float32),
        pltpu.VMEM((N,), jnp.float32),
        pltpu.VMEM((N,), jnp.float32),
        pltpu.VMEM((N,), jnp.float32),
        pltpu.VMEM((CPT, 3), jnp.float32),
        pltpu.VMEM((CPT, NSAMPLE), jnp.int32),
        pltpu.VMEM((3, CPT, NSAMPLE), jnp.float32),
    ],
)
def _ball_query_kernel(xyz, new_xyz, idx_out, gxyz_out,
                       pts, spv, x2b, y2b, z2b, ctrs, selbuf, xyzbuf):
    ci = lax.axis_index("c")
    si = lax.axis_index("s")
    wid = ci * NS + si
    b = wid // TPB
    m0 = (wid % TPB) * CPT

    pltpu.sync_copy(xyz.at[b, pl.ds(0, N), pl.ds(0, 3)], pts)
    pltpu.sync_copy(new_xyz.at[b, pl.ds(m0, CPT), pl.ds(0, 3)], ctrs)

    iot = lax.iota(jnp.int32, L)
    zz = jnp.zeros((L,), jnp.int32)
    one = zz + 1
    two = zz + 2
    for w in range(CPT // L):
        plsc.store_scatter(selbuf, [w * L + iot, zz], zz)

    def pre_round(i, carry):
        base = i * L
        bi = base + iot
        xs = plsc.load_gather(pts, [bi, zz])
        ys = plsc.load_gather(pts, [bi, one])
        zs = plsc.load_gather(pts, [bi, two])
        spv[pl.ds(base, L)] = (xs * xs + ys * ys) + zs * zs
        x2b[pl.ds(base, L)] = 2.0 * _bf16_round(xs)
        y2b[pl.ds(base, L)] = 2.0 * _bf16_round(ys)
        z2b[pl.ds(base, L)] = 2.0 * _bf16_round(zs)
        return carry

    lax.fori_loop(0, NCHUNK, pre_round, 0)

    def per_center(mi, carry):
        msp = jnp.full((L,), mi, jnp.int32)
        cx = plsc.load_gather(ctrs, [msp, zz])
        cy = plsc.load_gather(ctrs, [msp, one])
        cz = plsc.load_gather(ctrs, [msp, two])
        sc = (cx * cx + cy * cy) + cz * cz
        cxb = _bf16_round(cx)
        cyb = _bf16_round(cy)
        czb = _bf16_round(cz)

        def cond(st):
            i, cnt = st
            return jnp.logical_and(i < NCHUNK // 2, cnt < NSAMPLE)

        def body(st):
            i, cnt = st
            base = i * (2 * L)
            b2 = base + L
            d21 = (sc + spv[pl.ds(base, L)]) - (
                (cxb * x2b[pl.ds(base, L)] + cyb * y2b[pl.ds(base, L)])
                + czb * z2b[pl.ds(base, L)])
            d22 = (sc + spv[pl.ds(b2, L)]) - (
                (cxb * x2b[pl.ds(b2, L)] + cyb * y2b[pl.ds(b2, L)])
                + czb * z2b[pl.ds(b2, L)])
            m1 = d21 < R2
            m2 = d22 < R2
            p1 = plsc.all_reduce_population_count(m1)
            p2 = plsc.all_reduce_population_count(m2)
            cntv = jnp.full((L,), cnt, jnp.int32)
            inc1 = plsc.cumsum(m1.astype(jnp.int32))
            slot1 = (cntv + inc1) - 1
            wm1 = jnp.logical_and(m1, slot1 < NSAMPLE)
            plsc.store_scatter(selbuf, [msp, slot1], base + iot, mask=wm1)
            inc2 = plsc.cumsum(m2.astype(jnp.int32))
            slot2 = ((cntv + p1) + inc2) - 1
            wm2 = jnp.logical_and(m2, slot2 < NSAMPLE)
            plsc.store_scatter(selbuf, [msp, slot2], b2 + iot, mask=wm2)
            tot = p1 + p2
            return (i + 1, cnt + tot[0])

        _, cnt = lax.while_loop(cond, body, (jnp.int32(0), jnp.int32(0)))

        firstv = plsc.load_gather(selbuf, [msp, zz])
        cntv = jnp.full((L,), cnt, jnp.int32)
        for h in range(NSAMPLE // L):
            jv = h * L + iot
            cur = plsc.load_gather(selbuf, [msp, jv])
            selv = jnp.where(jv >= cntv, firstv, cur)
            plsc.store_scatter(selbuf, [msp, jv], selv)
            gx = plsc.load_gather(pts, [selv, zz]) - cx
            gy = plsc.load_gather(pts, [selv, one]) - cy
            gz = plsc.load_gather(pts, [selv, two]) - cz
            plsc.store_scatter(xyzbuf, [zz, msp, jv], gx)
            plsc.store_scatter(xyzbuf, [one, msp, jv], gy)
            plsc.store_scatter(xyzbuf, [two, msp, jv], gz)
        return carry

    lax.fori_loop(0, CPT, per_center, 0)

    pltpu.sync_copy(selbuf, idx_out.at[b, pl.ds(m0, CPT), pl.ds(0, NSAMPLE)])
    pltpu.sync_copy(xyzbuf,
                    gxyz_out.at[b, pl.ds(0, 3), pl.ds(m0, CPT), pl.ds(0, NSAMPLE)])


@functools.partial(
    pl.kernel,
    out_type=jax.ShapeDtypeStruct((B, 3 + C, NPOINT, NSAMPLE), jnp.float32),
    mesh=_mesh,
    compiler_params=_params,
    scratch_types=[
        pltpu.VMEM((NPOINT, NSAMPLE), jnp.int32),
        pltpu.VMEM((CB, N), jnp.float32),
        pltpu.VMEM((CB, RB, NSAMPLE), jnp.float32),
    ],
)
def _group_kernel(features, idxq, gxyz, out, idx_s, frows, obuf):
    ci = lax.axis_index("c")
    si = lax.axis_index("s")
    wid = ci * NS + si
    b = wid // TPB
    c0 = (wid % TPB) * CHT

    pltpu.sync_copy(idxq.at[b, pl.ds(0, NPOINT), pl.ds(0, NSAMPLE)], idx_s)

    @pl.when(wid % TPB == 0)
    def _copy_xyz():
        pltpu.sync_copy(
            gxyz.at[b, pl.ds(0, 3), pl.ds(0, NPOINT), pl.ds(0, NSAMPLE)],
            out.at[b, pl.ds(0, 3), pl.ds(0, NPOINT), pl.ds(0, NSAMPLE)])

    iot = lax.iota(jnp.int32, L)
    jsp = [jnp.full((L,), j, jnp.int32) for j in range(CB)]

    for cb in range(CHT // CB):
        c = c0 + cb * CB
        for j in range(CB):
            pltpu.sync_copy(features.at[b, c + j, pl.ds(0, N)],
                            frows.at[j, pl.ds(0, N)])

        for rb in range(NPOINT // RB):
            def per_row(ri, carry, rb=rb):
                rv = jnp.full((L,), rb * RB + ri, jnp.int32)
                riv = jnp.full((L,), ri, jnp.int32)
                for h in range(NSAMPLE // L):
                    cv = h * L + iot
                    iv = plsc.load_gather(idx_s, [rv, cv])
                    for j in range(CB):
                        vals = plsc.load_gather(frows, [jsp[j], iv])
                        plsc.store_scatter(obuf, [jsp[j], riv, cv], vals)
                return carry

            lax.fori_loop(0, RB, per_row, 0)
            for j in range(CB):
                pltpu.sync_copy(
                    obuf.at[j, pl.ds(0, RB), pl.ds(0, NSAMPLE)],
                    out.at[b, 3 + c + j, pl.ds(rb * RB, RB), pl.ds(0, NSAMPLE)])


def kernel(xyz, new_xyz, features):
    idxq, gxyz = _ball_query_kernel(xyz, new_xyz)
    return _group_kernel(features, idxq, gxyz)
